# baseline (speedup 1.0000x reference)
import functools

import jax
import jax.numpy as jnp
from jax import lax
from jax.experimental import pallas as pl
from jax.experimental.pallas import tpu as pltpu
from jax.experimental.pallas import tpu_sc as plsc

_B = 16384
_P = _B // 2
_POS = 1000000
_POS_PAD = 1000000
_MARGIN = 1.0
_BETA = 0.1
_LAMBDA = 1.0

_NS = 16
_CHUNK = _P // _NS
_L = 16
_NV = _CHUNK // _L
_GCH = 128
_NG = _CHUNK // _GCH
_TSL = (_POS // _NS) // 8 * 8
_TAIL = _POS - _NS * _TSL

_mesh = plsc.VectorSubcoreMesh(core_axis_name="c", subcore_axis_name="s", num_cores=1)


@functools.partial(
    pl.kernel,
    mesh=_mesh,
    out_type=jax.ShapeDtypeStruct((_L,), jnp.float32),
    scratch_types=[
        pltpu.VMEM((_CHUNK,), jnp.int32),
        pltpu.VMEM((_CHUNK,), jnp.float32),
        pltpu.VMEM((_CHUNK,), jnp.float32),
        pltpu.VMEM((_CHUNK,), jnp.float32),
        pltpu.VMEM((2 * _L,), jnp.float32),
        pltpu.VMEM_SHARED((_NS * 2 * _L,), jnp.float32),
        pltpu.VMEM((_NS * 2 * _L,), jnp.float32),
        pltpu.VMEM((_L,), jnp.float32),
        pltpu.VMEM_SHARED((_NS * _L,), jnp.float32),
        pltpu.VMEM((_NS * _L,), jnp.float32),
        pltpu.VMEM_SHARED((_POS,), jnp.float32),
        pltpu.VMEM((_TSL,), jnp.float32),
        pltpu.VMEM((8 * ((_TAIL + 7) // 8),), jnp.float32),
        pltpu.SemaphoreType.DMA,
        pltpu.SemaphoreType.DMA,
    ],
)
def _pauc_sc(y_pred_hbm, idx_hbm, u_pos_hbm, out_hbm,
             idx_v, g_v, ns_v, ps_v, stage_v, shared_es, all_v,
             stage_r, shared_r, rall_v, tbl_sh, tb_v, tail_v, sem, tsem):
    sid = lax.axis_index("s")
    base = sid * _CHUNK

    tcp = pltpu.async_copy(
        u_pos_hbm.at[pl.ds(sid * _TSL, _TSL)], tb_v, tsem,
    )
    tail_cp = pltpu.async_copy(
        u_pos_hbm.at[pl.ds(_NS * _TSL, _TAIL)],
        tail_v.at[pl.ds(0, _TAIL)],
        tsem,
    )

    pltpu.sync_copy(idx_hbm.at[pl.ds(base, _CHUNK)], idx_v)
    pltpu.sync_copy(y_pred_hbm.at[pl.ds(base, _CHUNK)], ns_v)
    pltpu.sync_copy(y_pred_hbm.at[pl.ds(_P + base, _CHUNK)], ps_v)

    acc_e = jnp.zeros((_L,), jnp.float32)
    acc_es = jnp.zeros((_L,), jnp.float32)
    for j in range(_NV):
        ns = ns_v[pl.ds(j * _L, _L)]
        ps = ps_v[pl.ds(j * _L, _L)]
        t = jnp.maximum(_MARGIN - (ps - ns), 0.0)
        s = t * t
        e = jnp.exp(s * (1.0 / _LAMBDA))
        acc_e = acc_e + e
        acc_es = acc_es + e * s
    stage_v[pl.ds(0, _L)] = acc_e
    stage_v[pl.ds(_L, _L)] = acc_es
    pltpu.sync_copy(stage_v, shared_es.at[pl.ds(sid * 2 * _L, 2 * _L)])
    tcp.wait()
    tail_cp.wait()
    pltpu.sync_copy(tb_v, tbl_sh.at[pl.ds(sid * _TSL, _TSL)])
    @pl.when(sid == 0)
    def _():
        pltpu.sync_copy(tail_v.at[pl.ds(0, _TAIL)],
                        tbl_sh.at[pl.ds(_NS * _TSL, _TAIL)])
    plsc.subcore_barrier()

    pltpu.sync_copy(shared_es, all_v)
    se = jnp.zeros((_L,), jnp.float32)
    ses = jnp.zeros((_L,), jnp.float32)
    for i in range(_NS):
        se = se + all_v[pl.ds(i * 2 * _L, _L)]
        ses = ses + all_v[pl.ds(i * 2 * _L + _L, _L)]
    m = se[0]
    a = ses[0]
    for l in range(1, _L):
        m = m + se[l]
        a = a + ses[l]
    m = m * (1.0 / _P)
    a = a * (1.0 / _P)

    gathers = [
        pltpu.async_copy(
            tbl_sh.at[idx_v.at[pl.ds(k * _GCH, _GCH)]],
            g_v.at[pl.ds(k * _GCH, _GCH)],
            sem,
        )
        for k in range(_NG)
    ]
    for c in gathers:
        c.wait()

    acc_r = jnp.zeros((_L,), jnp.float32)
    for j in range(_NV):
        g = g_v[pl.ds(j * _L, _L)]
        new = (1.0 - _BETA) * g + _BETA * m
        acc_r = acc_r + 1.0 / new
    stage_r[...] = acc_r
    pltpu.sync_copy(stage_r, shared_r.at[pl.ds(sid * _L, _L)])
    plsc.subcore_barrier()

    @pl.when(sid == 0)
    def _():
        pltpu.sync_copy(shared_r, rall_v)
        sr = jnp.zeros((_L,), jnp.float32)
        for i in range(_NS):
            sr = sr + rall_v[pl.ds(i * _L, _L)]
        r = sr[0]
        for l in range(1, _L):
            r = r + sr[l]
        r = r * (1.0 / _P)
        loss = a * r
        stage_r[...] = jnp.zeros((_L,), jnp.float32) + loss
        pltpu.sync_copy(stage_r, out_hbm)


def kernel(y_pred, y_true, index_p, u_pos):
    del y_true
    yp = y_pred.reshape(-1).astype(jnp.float32)
    idx = index_p.reshape(-1).astype(jnp.int32)
    up = u_pos.reshape(-1).astype(jnp.float32)
    out = _pauc_sc(yp, idx, up)
    return out[0]
